# baseline (speedup 1.0000x reference)
import functools
import jax
import jax.numpy as jnp
from jax import lax
from jax.experimental import pallas as pl
from jax.experimental.pallas import tpu as pltpu
from jax.experimental.pallas import tpu_sc as plsc

NC = 2
NS = 16
NW = NC * NS
HT = 25
BT = 32
TILES_PER_TEC = HT * BT // NW


def _gather_body(xt_hbm, tp_hbm, out_hbm, xidx_v, pidx_v, poff_v, rows0,
                 rows1, pk0, pk1, gsem0, gsem1, osem0, osem1):
    wid = lax.axis_index("s") * NC + lax.axis_index("c")
    base_t = wid * TILES_PER_TEC

    rows = (rows0, rows1)
    pks = (pk0, pk1)
    gsems = (gsem0, gsem1)
    osems = (osem0, osem1)

    @pl.loop(0, TILES_PER_TEC)
    def _tile(k):
        t = base_t + k
        ht = lax.shift_right_logical(t, 5)
        bt = lax.bitwise_and(t, BT - 1)

        pltpu.sync_copy(xt_hbm.at[pl.ds(ht * 8, 8), pl.ds(bt * 128, 128)],
                        xidx_v)
        for r8 in range(8):
            for g8 in range(8):
                v = xidx_v[r8, pl.ds(g8 * 16, 16)]
                pidx_v[r8, pl.ds(g8 * 16, 16)] = lax.shift_right_logical(v, 1)
                poff_v[r8, pl.ds(g8 * 16, 16)] = lax.shift_left(
                    lax.bitwise_and(v, 1), 6)

        def compact(r_dyn, buf):
            for b in range(128):
                q, s = b // 2, b % 2
                if b % 16 == 0:
                    pvv = poff_v[r_dyn, pl.ds(b, 16)]
                pv = pvv[b % 16]
                for c in range(4):
                    pks[buf][q, pl.ds(s * 64 + c * 16, 16)] = (
                        rows[buf][b, pl.ds(pv + c * 16, 16)])

        pltpu.async_copy(tp_hbm.at[pidx_v.at[0]], rows[0], gsem0)

        @pl.loop(0, 8, step=2)
        def _r(r):
            for sub in range(2):
                rr = r + sub
                buf = sub
                nbuf = 1 - sub
                nxt = lax.rem(rr + 1, 8)
                pltpu.async_copy(tp_hbm.at[pidx_v.at[nxt]], rows[nbuf],
                                 gsems[nbuf])
                pltpu.make_async_copy(
                    tp_hbm.at[pidx_v.at[0]], rows[buf], gsems[buf]).wait()

                @pl.when(jnp.logical_or(k > 0, r > 0))
                def _wait_prev():
                    pltpu.make_async_copy(
                        pks[buf], out_hbm.at[pl.ds(0, 64), :],
                        osems[buf]).wait()

                compact(rr, buf)
                h = ht * 8 + rr
                pltpu.async_copy(
                    pks[buf],
                    out_hbm.at[pl.ds(h * 2048 + bt * 64, 64), :],
                    osems[buf])

        pltpu.make_async_copy(
            tp_hbm.at[pidx_v.at[0]], rows[0], gsem0).wait()

    pltpu.make_async_copy(
        out_hbm.at[pl.ds(0, 64), :], pk0, osem0).wait()
    pltpu.make_async_copy(
        out_hbm.at[pl.ds(0, 64), :], pk1, osem1).wait()


def _sc_gather(xt, tp):
    mesh = plsc.VectorSubcoreMesh(core_axis_name="c", subcore_axis_name="s")
    return pl.kernel(
        _gather_body,
        out_type=jax.ShapeDtypeStruct((409600, 128), jnp.float32),
        mesh=mesh,
        scratch_types=[
            pltpu.VMEM((8, 128), jnp.int32),
            pltpu.VMEM((8, 128), jnp.int32),
            pltpu.VMEM((8, 128), jnp.int32),
            pltpu.VMEM((128, 128), jnp.float32),
            pltpu.VMEM((128, 128), jnp.float32),
            pltpu.VMEM((64, 128), jnp.float32),
            pltpu.VMEM((64, 128), jnp.float32),
            pltpu.SemaphoreType.DMA,
            pltpu.SemaphoreType.DMA,
            pltpu.SemaphoreType.DMA,
            pltpu.SemaphoreType.DMA,
        ],
        compiler_params=pltpu.CompilerParams(
            use_tc_tiling_on_sc=True, needs_layout_passes=False),
    )(xt, tp)


def _xpose_body(i_ref, o_ref):
    x = i_ref[...]
    x = x.reshape(64, 2, 64)
    y = jnp.transpose(x, (2, 0, 1))
    o_ref[0] = y.reshape(64, 128)


def _tc_xpose(mid):
    return pl.pallas_call(
        _xpose_body,
        grid=(200, 32),
        in_specs=[pl.BlockSpec((64, 128), lambda h, bt: (h * 32 + bt, 0))],
        out_specs=pl.BlockSpec((1, 64, 128), lambda h, bt: (h, 0, bt)),
        out_shape=jax.ShapeDtypeStruct((200, 64, 4096), jnp.float32),
    )(mid)


def kernel(x, table):
    xt = x.T
    tp = table.reshape(500000, 128)
    mid = _sc_gather(xt, tp)
    out = _tc_xpose(mid)
    return out.transpose(2, 0, 1)


# speedup vs baseline: 6.6163x; 6.6163x over previous
import jax
import jax.numpy as jnp
from jax import lax
from jax.experimental import pallas as pl
from jax.experimental.pallas import tpu as pltpu
from jax.experimental.pallas import tpu_sc as plsc

EMBED_DIM = 64
NC = 2
NS = 16
NW = NC * NS

PW = 128
NPAN = 999936 // PW
PAN_PER_TEC = -(-NPAN // NW)
PANW = PW * EMBED_DIM


def _xpose_table_body(tt_hbm, tail_hbm, t1d_hbm, blk0, blk1, orow0, orow1,
                      rsem0, rsem1, wsem0, wsem1):
    wid = lax.axis_index("s") * NC + lax.axis_index("c")

    lane = lax.iota(jnp.int32, 16)

    blks = (blk0, blk1)
    orows = (orow0, orow1)
    rsems = (rsem0, rsem1)
    wsems = (wsem0, wsem1)

    @pl.when(wid == 0)
    def _tail():
        pltpu.sync_copy(tail_hbm, blk0)
        for jl in range(64):
            for c in range(4):
                orow0[pl.ds(jl * EMBED_DIM + c * 16, 16)] = (
                    blk0[jl, pl.ds(c * 16, 16)])
        pltpu.sync_copy(orow0.at[pl.ds(0, 64 * EMBED_DIM)],
                        t1d_hbm.at[pl.ds(NPAN * PANW, 64 * EMBED_DIM)])

    def issue_read(slot, buf):
        pid = wid + slot * NW
        pltpu.async_copy(tt_hbm.at[:, pl.ds(pid * PW, PW)], blks[buf],
                         rsems[buf])

    @pl.when(wid < NPAN)
    def _first():
        issue_read(0, 0)

    @pl.loop(0, PAN_PER_TEC, step=2)
    def _panel2(slot0):
        for sub in range(2):
            slot = slot0 + sub
            buf = sub
            nbuf = 1 - sub
            pid = wid + slot * NW

            @pl.when(pid < NPAN)
            def _do(slot=slot, buf=buf, nbuf=nbuf, pid=pid):
                npid = wid + (slot + 1) * NW

                @pl.when(npid < NPAN)
                def _prefetch():
                    issue_read(slot + 1, nbuf)

                pltpu.make_async_copy(
                    tt_hbm.at[:, pl.ds(0, PW)], blks[buf],
                    rsems[buf]).wait()

                @pl.when(slot >= 2)
                def _wait_prev():
                    pltpu.make_async_copy(
                        orows[buf], t1d_hbm.at[pl.ds(0, PANW)],
                        wsems[buf]).wait()

                for t in range(PW):
                    jv = t + lane
                    jv = jv - jnp.where(jv >= PW, PW, 0)
                    jw = jv * EMBED_DIM + lane
                    for c in range(4):
                        val = plsc.load_gather(blks[buf],
                                               [lane + 16 * c, jv])
                        plsc.store_scatter(orows[buf], [jw + 16 * c], val)

                pltpu.async_copy(
                    orows[buf], t1d_hbm.at[pl.ds(pid * PANW, PANW)],
                    wsems[buf])

    @pl.when(wid < NPAN)
    def _drain0():
        pltpu.make_async_copy(
            t1d_hbm.at[pl.ds(0, PANW)], orow0, wsem0).wait()

    @pl.when(wid + NW < NPAN)
    def _drain1():
        pltpu.make_async_copy(
            t1d_hbm.at[pl.ds(0, PANW)], orow1, wsem1).wait()


def _xpose_table(tt, tailp):
    mesh = plsc.VectorSubcoreMesh(core_axis_name="c", subcore_axis_name="s")
    return pl.kernel(
        _xpose_table_body,
        out_type=jax.ShapeDtypeStruct((64000000,), jnp.float32),
        mesh=mesh,
        scratch_types=[
            pltpu.VMEM((64, PW), jnp.float32),
            pltpu.VMEM((64, PW), jnp.float32),
            pltpu.VMEM((PANW,), jnp.float32),
            pltpu.VMEM((PANW,), jnp.float32),
            pltpu.SemaphoreType.DMA,
            pltpu.SemaphoreType.DMA,
            pltpu.SemaphoreType.DMA,
            pltpu.SemaphoreType.DMA,
        ],
        compiler_params=pltpu.CompilerParams(
            use_tc_tiling_on_sc=True, needs_layout_passes=False),
    )(tt, tailp)


CB = 256
NBUF = 4
PIPE_D = 2


def _gather_body(x_hbm, table_hbm, out_hbm, idx_all, rows, gsems, ssems):
    n = x_hbm.shape[0]
    b_per_w = n // NW
    nchunks = b_per_w // CB
    nlaps = nchunks // NBUF
    wid = lax.axis_index("s") * NC + lax.axis_index("c")
    base = wid * b_per_w

    pltpu.sync_copy(x_hbm.at[pl.ds(base, b_per_w)], idx_all)

    def issue_gather(c, b):
        pltpu.async_copy(
            table_hbm.at[idx_all.at[pl.ds(c * CB, CB)]], rows[b], gsems[b])

    def wait_gather(b):
        pltpu.make_async_copy(
            table_hbm.at[idx_all.at[pl.ds(0, CB)]], rows[b], gsems[b]).wait()

    def issue_store(c, b):
        pltpu.async_copy(rows[b], out_hbm.at[pl.ds(base + c * CB, CB)],
                         ssems[b])

    def wait_store(b):
        pltpu.make_async_copy(
            rows[b], out_hbm.at[pl.ds(base, CB)], ssems[b]).wait()

    for c in range(PIPE_D):
        issue_gather(c, c % NBUF)
    for c in range(PIPE_D, NBUF):
        issue_gather(c, c % NBUF)
        wait_gather((c - PIPE_D) % NBUF)
        issue_store(c - PIPE_D, (c - PIPE_D) % NBUF)

    @pl.loop(1, nlaps)
    def _lap(g):
        for b in range(NBUF):
            c = g * NBUF + b
            wait_store(b)
            issue_gather(c, b)
            wait_gather((b - PIPE_D) % NBUF)
            issue_store(c - PIPE_D, (b - PIPE_D) % NBUF)

    for k in range(PIPE_D):
        b = (NBUF - PIPE_D + k) % NBUF
        wait_gather(b)
        issue_store(nchunks - PIPE_D + k, b)
    for b in range(NBUF):
        wait_store(b)


def _gather(flat_x, table_lin):
    n = flat_x.shape[0]
    mesh = plsc.VectorSubcoreMesh(core_axis_name="c", subcore_axis_name="s")
    return pl.kernel(
        _gather_body,
        out_type=jax.ShapeDtypeStruct((n, EMBED_DIM), jnp.float32),
        mesh=mesh,
        scratch_types=[
            pltpu.VMEM((n // NW,), jnp.int32),
            [pltpu.VMEM((CB, EMBED_DIM), jnp.float32) for _ in range(NBUF)],
            [pltpu.SemaphoreType.DMA for _ in range(NBUF)],
            [pltpu.SemaphoreType.DMA for _ in range(NBUF)],
        ],
        compiler_params=pltpu.CompilerParams(use_tc_tiling_on_sc=False),
    )(flat_x, table_lin)


def kernel(x, table):
    b, h = x.shape
    n = b * h
    tailp = jnp.pad(table[999936:], ((0, 0), (0, 64)))
    table_lin = _xpose_table(table.T, tailp).reshape(1000000, EMBED_DIM)
    out = _gather(x.reshape(n), table_lin)
    return out.reshape(b, h, EMBED_DIM)


# speedup vs baseline: 7.5714x; 1.1443x over previous
import jax
import jax.numpy as jnp
from jax import lax
from jax.experimental import pallas as pl
from jax.experimental.pallas import tpu as pltpu
from jax.experimental.pallas import tpu_sc as plsc

EMBED_DIM = 64
NC = 2
NS = 16
NW = NC * NS
CB = 256
NBUF = 4
PIPE_D = 2


def _gather_body(x_hbm, table_hbm, out_hbm, idx_all, rows, gsems, ssems):
    n = x_hbm.shape[0]
    b_per_w = n // NW
    nchunks = b_per_w // CB
    nlaps = nchunks // NBUF
    wid = lax.axis_index("s") * NC + lax.axis_index("c")
    base = wid * b_per_w

    pltpu.sync_copy(x_hbm.at[pl.ds(base, b_per_w)], idx_all)

    def issue_gather(c, b):
        pltpu.async_copy(
            table_hbm.at[idx_all.at[pl.ds(c * CB, CB)]], rows[b], gsems[b])

    def wait_gather(b):
        pltpu.make_async_copy(
            table_hbm.at[idx_all.at[pl.ds(0, CB)]], rows[b], gsems[b]).wait()

    def issue_store(c, b):
        pltpu.async_copy(rows[b], out_hbm.at[pl.ds(base + c * CB, CB)], ssems[b])

    def wait_store(b):
        pltpu.make_async_copy(
            rows[b], out_hbm.at[pl.ds(base, CB)], ssems[b]).wait()

    for c in range(PIPE_D):
        issue_gather(c, c % NBUF)
    for c in range(PIPE_D, NBUF):
        issue_gather(c, c % NBUF)
        wait_gather((c - PIPE_D) % NBUF)
        issue_store(c - PIPE_D, (c - PIPE_D) % NBUF)

    @pl.loop(1, nlaps)
    def _lap(g):
        for b in range(NBUF):
            c = g * NBUF + b
            wait_store(b)
            issue_gather(c, b)
            wait_gather((b - PIPE_D) % NBUF)
            issue_store(c - PIPE_D, (b - PIPE_D) % NBUF)

    for k in range(PIPE_D):
        b = (NBUF - PIPE_D + k) % NBUF
        wait_gather(b)
        issue_store(nchunks - PIPE_D + k, b)
    for b in range(NBUF):
        wait_store(b)


def kernel(x, table):
    b, h = x.shape
    n = b * h
    flat = x.reshape(n)
    mesh = plsc.VectorSubcoreMesh(core_axis_name="c", subcore_axis_name="s")
    out = pl.kernel(
        _gather_body,
        out_type=jax.ShapeDtypeStruct((n, EMBED_DIM), jnp.float32),
        mesh=mesh,
        scratch_types=[
            pltpu.VMEM((n // NW,), jnp.int32),
            [pltpu.VMEM((CB, EMBED_DIM), jnp.float32) for _ in range(NBUF)],
            [pltpu.SemaphoreType.DMA for _ in range(NBUF)],
            [pltpu.SemaphoreType.DMA for _ in range(NBUF)],
        ],
        compiler_params=pltpu.CompilerParams(use_tc_tiling_on_sc=False),
    )(flat, table)
    return out.reshape(b, h, EMBED_DIM)
